# Initial kernel scaffold; baseline (speedup 1.0000x reference)
import jax
import jax.numpy as jnp
from jax.experimental import pallas as pl


def kernel(x, edge_index, W1, b1, g1, bt1, W2, b2, g2, bt2):
    raise NotImplementedError("write your pallas kernel here")


# speedup vs baseline: 7.1863x; 7.1863x over previous
import functools

import jax
import jax.numpy as jnp
from jax import lax
from jax.experimental import pallas as pl
from jax.experimental.pallas import tpu as pltpu
from jax.experimental.pallas import tpu_sc as plsc

N = 10000
NP = 10240
D = 128
E = 320000
NC, NS = 2, 16
NW = NC * NS
CH = 128
K = 80
EPAD = NW * K * CH
ZR = NP // NS
BS = 1280
GRID = NP // BS


def _sc_mesh():
    return plsc.VectorSubcoreMesh(
        core_axis_name="c", subcore_axis_name="s",
        num_cores=NC, num_subcores=NS)


@functools.lru_cache(maxsize=None)
def _build_sc_deg():
    @functools.partial(
        pl.kernel,
        out_type=jax.ShapeDtypeStruct((NC, NP), jnp.float32),
        mesh=_sc_mesh(),
        scratch_types=[
            pltpu.VMEM((K, CH), jnp.int32),
            pltpu.VMEM((CH,), jnp.float32),
            pltpu.VMEM((ZR,), jnp.float32),
            pltpu.VMEM_SHARED((NP,), jnp.float32),
        ],
    )
    def deg_kernel(col_hbm, zeros1_hbm, ones_hbm, out_hbm,
                   colv, onesv, bounce, deg_sh):
        c = lax.axis_index("c")
        s = lax.axis_index("s")
        wid = s * NC + c
        pltpu.sync_copy(zeros1_hbm, deg_sh.at[pl.ds(s * ZR, ZR)])
        pltpu.sync_copy(ones_hbm, onesv)
        pltpu.sync_copy(col_hbm.at[wid], colv)
        plsc.subcore_barrier()

        def body(j, carry):
            pltpu.sync_copy(onesv, deg_sh.at[colv.at[j]], add=True)
            return carry

        lax.fori_loop(0, K, body, 0, unroll=False)
        plsc.subcore_barrier()
        pltpu.sync_copy(deg_sh.at[pl.ds(s * ZR, ZR)], bounce)
        pltpu.sync_copy(bounce, out_hbm.at[c, pl.ds(s * ZR, ZR)])

    return deg_kernel


def _sc_deg(colp, zeros1, ones):
    return _build_sc_deg()(colp, zeros1, ones)


@functools.lru_cache(maxsize=None)
def _build_sc_agg():
    @functools.partial(
        pl.kernel,
        out_type=jax.ShapeDtypeStruct((NC, NP, D), jnp.float32),
        mesh=_sc_mesh(),
        scratch_types=[
            pltpu.VMEM((K, CH), jnp.int32),
            pltpu.VMEM((K, CH), jnp.int32),
            pltpu.VMEM((CH, D), jnp.float32),
            pltpu.VMEM_SHARED((NP, D), jnp.float32),
            pltpu.SemaphoreType.DMA,
        ],
    )
    def agg_kernel(hs_hbm, row_hbm, col_hbm, zeros2_hbm, out_hbm,
                   rowv, colv, gbuf, agg_sh, sem):
        c = lax.axis_index("c")
        s = lax.axis_index("s")
        wid = s * NC + c
        pltpu.sync_copy(zeros2_hbm, agg_sh.at[pl.ds(s * ZR, ZR), :])
        pltpu.sync_copy(row_hbm.at[wid], rowv)
        pltpu.sync_copy(col_hbm.at[wid], colv)
        plsc.subcore_barrier()

        def body(j, carry):
            pltpu.async_copy(hs_hbm.at[rowv.at[j]], gbuf, sem).wait()
            pltpu.sync_copy(gbuf, agg_sh.at[colv.at[j]], add=True)
            return carry

        lax.fori_loop(0, K, body, 0, unroll=False)
        plsc.subcore_barrier()
        for k in range(ZR // CH):
            r = s * ZR + k * CH
            pltpu.sync_copy(agg_sh.at[pl.ds(r, CH), :], gbuf)
            pltpu.sync_copy(gbuf, out_hbm.at[c, pl.ds(r, CH), :])

    return agg_kernel


def _sc_agg(hs, rowp, colp, zeros2):
    return _build_sc_agg()(hs, rowp, colp, zeros2)


def _pre_body(x_ref, w_ref, dp_ref, dinv_ref, hs_ref):
    dsum = dp_ref[:, 0:1] + dp_ref[:, 1:2] + 1.0
    dinv = lax.rsqrt(dsum)
    h = jax.lax.dot_general(
        x_ref[...], w_ref[...], (((1,), (0,)), ((), ())),
        preferred_element_type=jnp.float32, precision=lax.Precision.HIGHEST)
    dinv_ref[...] = dinv
    hs_ref[...] = h * dinv


def _tc_pre(x_p, W, dp_t):
    return pl.pallas_call(
        _pre_body,
        grid=(GRID,),
        in_specs=[
            pl.BlockSpec((BS, D), lambda i: (i, 0)),
            pl.BlockSpec((D, D), lambda i: (0, 0)),
            pl.BlockSpec((BS, 2), lambda i: (i, 0)),
        ],
        out_specs=[
            pl.BlockSpec((BS, 1), lambda i: (i, 0)),
            pl.BlockSpec((BS, D), lambda i: (i, 0)),
        ],
        out_shape=[
            jax.ShapeDtypeStruct((NP, 1), jnp.float32),
            jax.ShapeDtypeStruct((NP, D), jnp.float32),
        ],
    )(x_p, W, dp_t)


def _stats_body(p_ref, hs_ref, dinv_ref, b_ref, t_ref, st_ref):
    i = pl.program_id(0)
    t = (p_ref[0] + p_ref[1] + hs_ref[...]) * dinv_ref[...] + b_ref[...]
    t_ref[...] = t
    rid = i * BS + lax.broadcasted_iota(jnp.int32, (BS, 1), 0)
    tm = jnp.where(rid < N, t, 0.0)

    @pl.when(i == 0)
    def _():
        st_ref[...] = jnp.zeros_like(st_ref)

    st_ref[0:1, :] += jnp.sum(tm, axis=0, keepdims=True)
    st_ref[1:2, :] += jnp.sum(tm * tm, axis=0, keepdims=True)


def _tc_stats(p, hs, dinv, b):
    return pl.pallas_call(
        _stats_body,
        grid=(GRID,),
        in_specs=[
            pl.BlockSpec((NC, BS, D), lambda i: (0, i, 0)),
            pl.BlockSpec((BS, D), lambda i: (i, 0)),
            pl.BlockSpec((BS, 1), lambda i: (i, 0)),
            pl.BlockSpec((1, D), lambda i: (0, 0)),
        ],
        out_specs=[
            pl.BlockSpec((BS, D), lambda i: (i, 0)),
            pl.BlockSpec((8, D), lambda i: (0, 0)),
        ],
        out_shape=[
            jax.ShapeDtypeStruct((NP, D), jnp.float32),
            jax.ShapeDtypeStruct((8, D), jnp.float32),
        ],
    )(p, hs, dinv, b)


def _mid_body(t_ref, st_ref, g_ref, bt_ref, w_ref, dinv_ref, hs2_ref):
    mu = st_ref[0:1, :] * (1.0 / N)
    var = st_ref[1:2, :] * (1.0 / N) - mu * mu
    y = (t_ref[...] - mu) * lax.rsqrt(var + 1e-5) * g_ref[...] + bt_ref[...]
    r = jnp.maximum(y, 0.0)
    h = jax.lax.dot_general(
        r, w_ref[...], (((1,), (0,)), ((), ())),
        preferred_element_type=jnp.float32, precision=lax.Precision.HIGHEST)
    hs2_ref[...] = h * dinv_ref[...]


def _tc_mid(t, st, g, bt, W, dinv):
    return pl.pallas_call(
        _mid_body,
        grid=(GRID,),
        in_specs=[
            pl.BlockSpec((BS, D), lambda i: (i, 0)),
            pl.BlockSpec((8, D), lambda i: (0, 0)),
            pl.BlockSpec((1, D), lambda i: (0, 0)),
            pl.BlockSpec((1, D), lambda i: (0, 0)),
            pl.BlockSpec((D, D), lambda i: (0, 0)),
            pl.BlockSpec((BS, 1), lambda i: (i, 0)),
        ],
        out_specs=pl.BlockSpec((BS, D), lambda i: (i, 0)),
        out_shape=jax.ShapeDtypeStruct((NP, D), jnp.float32),
    )(t, st, g, bt, W, dinv)


def _post_body(u_ref, st_ref, g_ref, bt_ref, x_ref, o_ref):
    mu = st_ref[0:1, :] * (1.0 / N)
    var = st_ref[1:2, :] * (1.0 / N) - mu * mu
    y = (u_ref[...] - mu) * lax.rsqrt(var + 1e-5) * g_ref[...] + bt_ref[...]
    o_ref[...] = jnp.maximum(y + x_ref[...], 0.0)


def _tc_post(u, st, g, bt, x_p):
    return pl.pallas_call(
        _post_body,
        grid=(GRID,),
        in_specs=[
            pl.BlockSpec((BS, D), lambda i: (i, 0)),
            pl.BlockSpec((8, D), lambda i: (0, 0)),
            pl.BlockSpec((1, D), lambda i: (0, 0)),
            pl.BlockSpec((1, D), lambda i: (0, 0)),
            pl.BlockSpec((BS, D), lambda i: (i, 0)),
        ],
        out_specs=pl.BlockSpec((BS, D), lambda i: (i, 0)),
        out_shape=jax.ShapeDtypeStruct((NP, D), jnp.float32),
    )(u, st, g, bt, x_p)


def kernel(x, edge_index, W1, b1, g1, bt1, W2, b2, g2, bt2):
    row = edge_index[0]
    col = edge_index[1]
    pad = EPAD - E
    rowp = jnp.concatenate(
        [row, jnp.zeros((pad,), jnp.int32)]).reshape(NW, K, CH)
    colp = jnp.concatenate(
        [col, jnp.full((pad,), N, jnp.int32)]).reshape(NW, K, CH)
    x_p = jnp.pad(x, ((0, NP - N), (0, 0)))
    zeros1 = jnp.zeros((ZR,), jnp.float32)
    zeros2 = jnp.zeros((ZR, D), jnp.float32)
    ones = jnp.ones((CH,), jnp.float32)
    b1r, g1r, bt1r = b1.reshape(1, D), g1.reshape(1, D), bt1.reshape(1, D)
    b2r, g2r, bt2r = b2.reshape(1, D), g2.reshape(1, D), bt2.reshape(1, D)

    dp = _sc_deg(colp, zeros1, ones)
    dinv, hs1 = _tc_pre(x_p, W1, dp.T)

    p1 = _sc_agg(hs1, rowp, colp, zeros2)
    t, st1 = _tc_stats(p1, hs1, dinv, b1r)
    hs2 = _tc_mid(t, st1, g1r, bt1r, W2, dinv)

    p2 = _sc_agg(hs2, rowp, colp, zeros2)
    u, st2 = _tc_stats(p2, hs2, dinv, b2r)
    outp = _tc_post(u, st2, g2r, bt2r, x_p)
    return outp[:N]


# speedup vs baseline: 8.2585x; 1.1492x over previous
import functools

import jax
import jax.numpy as jnp
from jax import lax
from jax.experimental import pallas as pl
from jax.experimental.pallas import tpu as pltpu
from jax.experimental.pallas import tpu_sc as plsc

N = 10000
NP = 10240
D = 128
E = 320000
NC, NS = 2, 16
NW = NC * NS
CH = 64
K = 160
EPAD = NW * K * CH
ZR = NP // NS
BS = 1280
GRID = NP // BS


def _sc_mesh():
    return plsc.VectorSubcoreMesh(
        core_axis_name="c", subcore_axis_name="s",
        num_cores=NC, num_subcores=NS)


@functools.lru_cache(maxsize=None)
def _build_sc_deg():
    @functools.partial(
        pl.kernel,
        out_type=jax.ShapeDtypeStruct((NC, NP), jnp.float32),
        mesh=_sc_mesh(),
        scratch_types=[
            pltpu.VMEM((K, CH), jnp.int32),
            pltpu.VMEM((CH,), jnp.float32),
            pltpu.VMEM((ZR,), jnp.float32),
            pltpu.VMEM_SHARED((NP,), jnp.float32),
        ],
    )
    def deg_kernel(col_hbm, zeros1_hbm, ones_hbm, out_hbm,
                   colv, onesv, bounce, deg_sh):
        c = lax.axis_index("c")
        s = lax.axis_index("s")
        wid = s * NC + c
        pltpu.sync_copy(zeros1_hbm, deg_sh.at[pl.ds(s * ZR, ZR)])
        pltpu.sync_copy(ones_hbm, onesv)
        pltpu.sync_copy(col_hbm.at[wid], colv)
        plsc.subcore_barrier()

        def body(j, carry):
            pltpu.sync_copy(onesv, deg_sh.at[colv.at[j]], add=True)
            return carry

        lax.fori_loop(0, K, body, 0, unroll=False)
        plsc.subcore_barrier()
        pltpu.sync_copy(deg_sh.at[pl.ds(s * ZR, ZR)], bounce)
        pltpu.sync_copy(bounce, out_hbm.at[c, pl.ds(s * ZR, ZR)])

    return deg_kernel


def _sc_deg(colp, zeros1, ones):
    return _build_sc_deg()(colp, zeros1, ones)


@functools.lru_cache(maxsize=None)
def _build_sc_agg():
    @functools.partial(
        pl.kernel,
        out_type=jax.ShapeDtypeStruct((NC, NP, D), jnp.float32),
        mesh=_sc_mesh(),
        scratch_types=[
            pltpu.VMEM((K // 2, CH), jnp.int32),
            pltpu.VMEM((K // 2, CH), jnp.int32),
            pltpu.VMEM((CH, D), jnp.float32),
            pltpu.VMEM((CH, D), jnp.float32),
            pltpu.VMEM_SHARED((NP, D), jnp.float32),
            pltpu.SemaphoreType.DMA,
            pltpu.SemaphoreType.DMA,
        ],
    )
    def agg_kernel(hs_hbm, row_hbm, col_hbm, zeros2_hbm, out_hbm,
                   rowv, colv, g0, g1, agg_sh, sem0, sem1):
        c = lax.axis_index("c")
        s = lax.axis_index("s")
        wid = s * NC + c
        K2 = K // 2
        pltpu.sync_copy(zeros2_hbm, agg_sh.at[pl.ds(s * ZR, ZR), :])
        plsc.subcore_barrier()

        def body(g, carry):
            j0 = 2 * g
            pltpu.async_copy(hs_hbm.at[rowv.at[j0 + 1]], g1, sem1)
            pltpu.make_async_copy(hs_hbm.at[rowv.at[j0]], g0, sem0).wait()
            pltpu.sync_copy(g0, agg_sh.at[colv.at[j0]], add=True)
            jn = jnp.minimum(j0 + 2, K2 - 1)
            pltpu.async_copy(hs_hbm.at[rowv.at[jn]], g0, sem0)
            pltpu.make_async_copy(hs_hbm.at[rowv.at[j0 + 1]], g1, sem1).wait()
            pltpu.sync_copy(g1, agg_sh.at[colv.at[j0 + 1]], add=True)
            return carry

        for h in range(2):
            pltpu.sync_copy(row_hbm.at[wid, pl.ds(h * K2, K2)], rowv)
            pltpu.sync_copy(col_hbm.at[wid, pl.ds(h * K2, K2)], colv)
            pltpu.async_copy(hs_hbm.at[rowv.at[0]], g0, sem0)
            lax.fori_loop(0, K2 // 2, body, 0, unroll=False)
            pltpu.make_async_copy(hs_hbm.at[rowv.at[K2 - 1]], g0, sem0).wait()
        plsc.subcore_barrier()
        for k in range(ZR // CH):
            r = s * ZR + k * CH
            pltpu.sync_copy(agg_sh.at[pl.ds(r, CH), :], g0)
            pltpu.sync_copy(g0, out_hbm.at[c, pl.ds(r, CH), :])

    return agg_kernel


def _sc_agg(hs, rowp, colp, zeros2):
    return _build_sc_agg()(hs, rowp, colp, zeros2)


def _pre_body(x_ref, w_ref, dp_ref, dinv_ref, hs_ref):
    dsum = dp_ref[:, 0:1] + dp_ref[:, 1:2] + 1.0
    dinv = lax.rsqrt(dsum)
    h = jax.lax.dot_general(
        x_ref[...], w_ref[...], (((1,), (0,)), ((), ())),
        preferred_element_type=jnp.float32, precision=lax.Precision.HIGHEST)
    dinv_ref[...] = dinv
    hs_ref[...] = h * dinv


def _tc_pre(x_p, W, dp_t):
    return pl.pallas_call(
        _pre_body,
        grid=(GRID,),
        in_specs=[
            pl.BlockSpec((BS, D), lambda i: (i, 0)),
            pl.BlockSpec((D, D), lambda i: (0, 0)),
            pl.BlockSpec((BS, 2), lambda i: (i, 0)),
        ],
        out_specs=[
            pl.BlockSpec((BS, 1), lambda i: (i, 0)),
            pl.BlockSpec((BS, D), lambda i: (i, 0)),
        ],
        out_shape=[
            jax.ShapeDtypeStruct((NP, 1), jnp.float32),
            jax.ShapeDtypeStruct((NP, D), jnp.float32),
        ],
    )(x_p, W, dp_t)


def _stats_body(p_ref, hs_ref, dinv_ref, b_ref, t_ref, st_ref):
    i = pl.program_id(0)
    t = (p_ref[0] + p_ref[1] + hs_ref[...]) * dinv_ref[...] + b_ref[...]
    t_ref[...] = t
    rid = i * BS + lax.broadcasted_iota(jnp.int32, (BS, 1), 0)
    tm = jnp.where(rid < N, t, 0.0)

    @pl.when(i == 0)
    def _():
        st_ref[...] = jnp.zeros_like(st_ref)

    st_ref[0:1, :] += jnp.sum(tm, axis=0, keepdims=True)
    st_ref[1:2, :] += jnp.sum(tm * tm, axis=0, keepdims=True)


def _tc_stats(p, hs, dinv, b):
    return pl.pallas_call(
        _stats_body,
        grid=(GRID,),
        in_specs=[
            pl.BlockSpec((NC, BS, D), lambda i: (0, i, 0)),
            pl.BlockSpec((BS, D), lambda i: (i, 0)),
            pl.BlockSpec((BS, 1), lambda i: (i, 0)),
            pl.BlockSpec((1, D), lambda i: (0, 0)),
        ],
        out_specs=[
            pl.BlockSpec((BS, D), lambda i: (i, 0)),
            pl.BlockSpec((8, D), lambda i: (0, 0)),
        ],
        out_shape=[
            jax.ShapeDtypeStruct((NP, D), jnp.float32),
            jax.ShapeDtypeStruct((8, D), jnp.float32),
        ],
    )(p, hs, dinv, b)


def _mid_body(t_ref, st_ref, g_ref, bt_ref, w_ref, dinv_ref, hs2_ref):
    mu = st_ref[0:1, :] * (1.0 / N)
    var = st_ref[1:2, :] * (1.0 / N) - mu * mu
    y = (t_ref[...] - mu) * lax.rsqrt(var + 1e-5) * g_ref[...] + bt_ref[...]
    r = jnp.maximum(y, 0.0)
    h = jax.lax.dot_general(
        r, w_ref[...], (((1,), (0,)), ((), ())),
        preferred_element_type=jnp.float32, precision=lax.Precision.HIGHEST)
    hs2_ref[...] = h * dinv_ref[...]


def _tc_mid(t, st, g, bt, W, dinv):
    return pl.pallas_call(
        _mid_body,
        grid=(GRID,),
        in_specs=[
            pl.BlockSpec((BS, D), lambda i: (i, 0)),
            pl.BlockSpec((8, D), lambda i: (0, 0)),
            pl.BlockSpec((1, D), lambda i: (0, 0)),
            pl.BlockSpec((1, D), lambda i: (0, 0)),
            pl.BlockSpec((D, D), lambda i: (0, 0)),
            pl.BlockSpec((BS, 1), lambda i: (i, 0)),
        ],
        out_specs=pl.BlockSpec((BS, D), lambda i: (i, 0)),
        out_shape=jax.ShapeDtypeStruct((NP, D), jnp.float32),
    )(t, st, g, bt, W, dinv)


def _post_body(u_ref, st_ref, g_ref, bt_ref, x_ref, o_ref):
    mu = st_ref[0:1, :] * (1.0 / N)
    var = st_ref[1:2, :] * (1.0 / N) - mu * mu
    y = (u_ref[...] - mu) * lax.rsqrt(var + 1e-5) * g_ref[...] + bt_ref[...]
    o_ref[...] = jnp.maximum(y + x_ref[...], 0.0)


def _tc_post(u, st, g, bt, x_p):
    return pl.pallas_call(
        _post_body,
        grid=(GRID,),
        in_specs=[
            pl.BlockSpec((BS, D), lambda i: (i, 0)),
            pl.BlockSpec((8, D), lambda i: (0, 0)),
            pl.BlockSpec((1, D), lambda i: (0, 0)),
            pl.BlockSpec((1, D), lambda i: (0, 0)),
            pl.BlockSpec((BS, D), lambda i: (i, 0)),
        ],
        out_specs=pl.BlockSpec((BS, D), lambda i: (i, 0)),
        out_shape=jax.ShapeDtypeStruct((NP, D), jnp.float32),
    )(u, st, g, bt, x_p)


def kernel(x, edge_index, W1, b1, g1, bt1, W2, b2, g2, bt2):
    row = edge_index[0]
    col = edge_index[1]
    padw = (EPAD - E) // NW
    padrow = jnp.zeros((NW, padw), jnp.int32)
    padcol = jnp.broadcast_to(
        N + jnp.arange(padw, dtype=jnp.int32), (NW, padw))
    rowp = jnp.concatenate(
        [row.reshape(NW, E // NW), padrow], axis=1).reshape(NW, K, CH)
    colp = jnp.concatenate(
        [col.reshape(NW, E // NW), padcol], axis=1).reshape(NW, K, CH)
    x_p = jnp.pad(x, ((0, NP - N), (0, 0)))
    zeros1 = jnp.zeros((ZR,), jnp.float32)
    zeros2 = jnp.zeros((ZR, D), jnp.float32)
    ones = jnp.ones((CH,), jnp.float32)
    b1r, g1r, bt1r = b1.reshape(1, D), g1.reshape(1, D), bt1.reshape(1, D)
    b2r, g2r, bt2r = b2.reshape(1, D), g2.reshape(1, D), bt2.reshape(1, D)

    dp = _sc_deg(colp, zeros1, ones)
    dinv, hs1 = _tc_pre(x_p, W1, dp.T)

    p1 = _sc_agg(hs1, rowp, colp, zeros2)
    t, st1 = _tc_stats(p1, hs1, dinv, b1r)
    hs2 = _tc_mid(t, st1, g1r, bt1r, W2, dinv)

    p2 = _sc_agg(hs2, rowp, colp, zeros2)
    u, st2 = _tc_stats(p2, hs2, dinv, b2r)
    outp = _tc_post(u, st2, g2r, bt2r, x_p)
    return outp[:N]
